# baseline (speedup 1.0000x reference)
import jax
import jax.numpy as jnp
from jax.experimental import pallas as pl
from jax.experimental.pallas import tpu as pltpu

_WL = 8
_LOSS_THR = 1.0
_C = 32
_H = 512
_W = 512
_NB = _W // _WL
_R = _C * _WL


def _minmax_body(x_ref, mx_ref, mn_ref):
    i = pl.program_id(0)
    t = x_ref[...]
    xl = jnp.concatenate(
        [jnp.zeros((t.shape[0], t.shape[1], 1), t.dtype), t[:, :, :-1]], axis=2)
    xd = t - xl
    m = jnp.max(xd)
    n = jnp.min(xd)

    @pl.when(i == 0)
    def _():
        mx_ref[0, 0] = m
        mn_ref[0, 0] = n

    @pl.when(i > 0)
    def _():
        mx_ref[0, 0] = jnp.maximum(mx_ref[0, 0], m)
        mn_ref[0, 0] = jnp.minimum(mn_ref[0, 0], n)


_UNROLL = 2


def _fit_stripe(t, mn, sc, isc, lsb, ilsb, S, SQ, E, PT, P):
    xl = jnp.concatenate(
        [jnp.zeros((_C, _WL, 1), t.dtype), t[:, :, :-1]], axis=2)
    xd = t - xl
    v = jnp.round((xd - mn) * sc) * isc + mn

    a1 = v[0]
    a2 = v[1]

    def bsum(z):
        zs = jnp.dot(PT, z.reshape(_R, _W), preferred_element_type=jnp.float32)
        return jnp.dot(zs, S, preferred_element_type=jnp.float32)

    B3 = bsum(v)
    B1 = bsum(v * a1[None])
    B2 = bsum(v * a2[None])
    s1 = B3[0]
    s2 = B3[1]
    s11 = B1[0]
    s12 = B1[1]
    s22 = B2[1]
    n = jnp.float32(_WL * _WL)

    det = (s11 * (s22 * n - s2 * s2)
           - s12 * (s12 * n - s2 * s1)
           + s1 * (s12 * s2 - s22 * s1))
    sing = det == 0.0
    idet = 1.0 / jnp.where(sing, 1.0, det)
    i00 = jnp.where(sing, 1.0, (s22 * n - s2 * s2) * idet)
    i01 = jnp.where(sing, 0.0, (s1 * s2 - s12 * n) * idet)
    i02 = jnp.where(sing, 0.0, (s12 * s2 - s22 * s1) * idet)
    i11 = jnp.where(sing, 1.0, (s11 * n - s1 * s1) * idet)
    i12 = jnp.where(sing, 0.0, (s12 * s1 - s11 * s2) * idet)
    i22 = jnp.where(sing, 1.0, (s11 * s22 - s12 * s12) * idet)

    def san(c):
        return jnp.nan_to_num(c, nan=1e15, posinf=1e15, neginf=-1e15)

    c0 = san(i00[None] * B1 + i01[None] * B2 + i02[None] * B3)
    c1 = san(i01[None] * B1 + i11[None] * B2 + i12[None] * B3)
    c2 = san(i02[None] * B1 + i12[None] * B2 + i22[None] * B3)

    def expand(ck):
        cw = jnp.dot(ck, E, preferred_element_type=jnp.float32)
        return jnp.dot(P, cw,
                       preferred_element_type=jnp.float32).reshape(_C, _WL, _W)

    r = expand(c0) * a1[None] + expand(c1) * a2[None] + expand(c2)
    r1 = jnp.round(r * ilsb) * lsb

    sq = (v - r1) ** 2
    L = jnp.dot(jnp.dot(PT, sq.reshape(_R, _W),
                        preferred_element_type=jnp.float32), S,
                preferred_element_type=jnp.float32)

    q = (jnp.abs(a1 - jnp.roll(a1, -1, axis=1))
         + jnp.abs(a2 - jnp.roll(a2, -1, axis=1)))
    Q = jnp.dot(q.sum(axis=0)[None, :], SQ,
                preferred_element_type=jnp.float32)
    L = jnp.where(Q == 0.0, _LOSS_THR + 1.0, L)

    LE = jnp.dot(L, E, preferred_element_type=jnp.float32)
    take_fit = LE <= _LOSS_THR
    rr = jnp.where(take_fit[:, None, :], r1, v)
    return rr + xl


def _fit_body(s_ref, x_ref, S_ref, SQ_ref, E_ref, PT_ref, P_ref, o_ref):
    mn = s_ref[0, 0]
    sc = s_ref[0, 1]
    isc = s_ref[0, 2]
    lsb = s_ref[0, 3]
    ilsb = s_ref[0, 4]
    S = S_ref[...]
    SQ = SQ_ref[...]
    E = E_ref[...]
    PT = PT_ref[...]
    P = P_ref[...]

    for s in range(_UNROLL):
        t = x_ref[:, s * _WL:(s + 1) * _WL, :]
        o_ref[:, s * _WL:(s + 1) * _WL, :] = _fit_stripe(
            t, mn, sc, isc, lsb, ilsb, S, SQ, E, PT, P)


def _constants():
    w = jnp.arange(_W, dtype=jnp.int32)
    b = jnp.arange(_NB, dtype=jnp.int32)
    S = (w[:, None] // _WL == b[None, :]).astype(jnp.float32)
    SQ = S * (w[:, None] % _WL != _WL - 1).astype(jnp.float32)
    E = S.T
    r = jnp.arange(_R, dtype=jnp.int32)
    c = jnp.arange(_C, dtype=jnp.int32)
    PT = (r[None, :] // _WL == c[:, None]).astype(jnp.float32)
    return S, SQ, E, PT, PT.T


def kernel(x):
    x2 = x[0]

    mx, mn = pl.pallas_call(
        _minmax_body,
        grid=(8,),
        in_specs=[pl.BlockSpec((_C, _H // 8, _W), lambda i: (0, i, 0))],
        out_specs=[
            pl.BlockSpec((1, 1), lambda i: (0, 0), memory_space=pltpu.SMEM),
            pl.BlockSpec((1, 1), lambda i: (0, 0), memory_space=pltpu.SMEM),
        ],
        out_shape=[
            jax.ShapeDtypeStruct((1, 1), jnp.float32),
            jax.ShapeDtypeStruct((1, 1), jnp.float32),
        ],
    )(x2)
    mx = mx[0, 0]
    mn = mn[0, 0]

    scale = (2.0 ** 16 - 1.0) / (mx - mn)
    iscale = 1.0 / scale
    lsb = 2.0 ** (jnp.round(jnp.log2(mx / 2.0 ** 15)) + 1.0)
    ilsb = 1.0 / lsb
    scalars = jnp.stack([mn, scale, iscale, lsb, ilsb,
                         jnp.float32(0), jnp.float32(0), jnp.float32(0)])
    scalars = scalars.astype(jnp.float32).reshape(1, 8)

    S, SQ, E, PT, P = _constants()
    inv = lambda i: (0, 0)

    out = pl.pallas_call(
        _fit_body,
        grid=(_H // (_WL * _UNROLL),),
        in_specs=[
            pl.BlockSpec(memory_space=pltpu.SMEM),
            pl.BlockSpec((_C, _WL * _UNROLL, _W), lambda i: (0, i, 0)),
            pl.BlockSpec((_W, _NB), inv),
            pl.BlockSpec((_W, _NB), inv),
            pl.BlockSpec((_NB, _W), inv),
            pl.BlockSpec((_C, _R), inv),
            pl.BlockSpec((_R, _C), inv),
        ],
        out_specs=pl.BlockSpec((_C, _WL * _UNROLL, _W), lambda i: (0, i, 0)),
        out_shape=jax.ShapeDtypeStruct((_C, _H, _W), jnp.float32),
    )(scalars, x2, S, SQ, E, PT, P)
    return out[None]


# speedup vs baseline: 1.3657x; 1.3657x over previous
import jax
import jax.numpy as jnp
from jax.experimental import pallas as pl
from jax.experimental.pallas import tpu as pltpu

_WL = 8
_LOSS_THR = 1.0
_C = 32
_H = 512
_W = 512
_NB = _W // _WL
_R = _C * _WL


def _minmax_body(x_ref, mx_ref, mn_ref):
    i = pl.program_id(0)
    t = x_ref[...]
    xl = jnp.concatenate(
        [jnp.zeros((t.shape[0], t.shape[1], 1), t.dtype), t[:, :, :-1]], axis=2)
    xd = t - xl
    m = jnp.max(xd)
    n = jnp.min(xd)

    @pl.when(i == 0)
    def _():
        mx_ref[0, 0] = m
        mn_ref[0, 0] = n

    @pl.when(i > 0)
    def _():
        mx_ref[0, 0] = jnp.maximum(mx_ref[0, 0], m)
        mn_ref[0, 0] = jnp.minimum(mn_ref[0, 0], n)


_UNROLL = 4


def _fit_stripe(t, mn, sc, isc, lsb, ilsb, S, SQ, E, PT):
    xl = jnp.concatenate(
        [jnp.zeros((_C, _WL, 1), t.dtype), t[:, :, :-1]], axis=2)
    xd = t - xl
    v = jnp.round((xd - mn) * sc) * isc + mn

    a1 = v[0]
    a2 = v[1]

    def bsum(z):
        return jnp.dot(z.sum(axis=1), S, preferred_element_type=jnp.float32)

    B3 = bsum(v)
    B1 = bsum(v * a1[None])
    B2 = bsum(v * a2[None])
    s1 = B3[0]
    s2 = B3[1]
    s11 = B1[0]
    s12 = B1[1]
    s22 = B2[1]
    n = jnp.float32(_WL * _WL)

    det = (s11 * (s22 * n - s2 * s2)
           - s12 * (s12 * n - s2 * s1)
           + s1 * (s12 * s2 - s22 * s1))
    sing = det == 0.0
    idet = 1.0 / jnp.where(sing, 1.0, det)
    i00 = jnp.where(sing, 1.0, (s22 * n - s2 * s2) * idet)
    i01 = jnp.where(sing, 0.0, (s1 * s2 - s12 * n) * idet)
    i02 = jnp.where(sing, 0.0, (s12 * s2 - s22 * s1) * idet)
    i11 = jnp.where(sing, 1.0, (s11 * n - s1 * s1) * idet)
    i12 = jnp.where(sing, 0.0, (s12 * s1 - s11 * s2) * idet)
    i22 = jnp.where(sing, 1.0, (s11 * s22 - s12 * s12) * idet)

    def san(c):
        return jnp.nan_to_num(c, nan=1e15, posinf=1e15, neginf=-1e15)

    c0 = san(i00[None] * B1 + i01[None] * B2 + i02[None] * B3)
    c1 = san(i01[None] * B1 + i11[None] * B2 + i12[None] * B3)
    c2 = san(i02[None] * B1 + i12[None] * B2 + i22[None] * B3)

    C0 = jnp.dot(c0, E, preferred_element_type=jnp.float32)
    C1 = jnp.dot(c1, E, preferred_element_type=jnp.float32)
    C2 = jnp.dot(c2, E, preferred_element_type=jnp.float32)

    r = C0[:, None, :] * a1[None] + C1[:, None, :] * a2[None] + C2[:, None, :]
    r1 = jnp.round(r * ilsb) * lsb

    sq = (v - r1) ** 2
    L = jnp.dot(sq.sum(axis=1), S,
                preferred_element_type=jnp.float32)

    q = (jnp.abs(a1 - jnp.roll(a1, -1, axis=1))
         + jnp.abs(a2 - jnp.roll(a2, -1, axis=1)))
    Q = jnp.dot(q.sum(axis=0)[None, :], SQ,
                preferred_element_type=jnp.float32)
    L = jnp.where(Q == 0.0, _LOSS_THR + 1.0, L)

    LE = jnp.dot(L, E, preferred_element_type=jnp.float32)
    take_fit = LE <= _LOSS_THR
    rr = jnp.where(take_fit[:, None, :], r1, v)
    return rr + xl


def _fit_body(s_ref, x_ref, S_ref, SQ_ref, E_ref, PT_ref, o_ref):
    mn = s_ref[0, 0]
    sc = s_ref[0, 1]
    isc = s_ref[0, 2]
    lsb = s_ref[0, 3]
    ilsb = s_ref[0, 4]
    S = S_ref[...]
    SQ = SQ_ref[...]
    E = E_ref[...]
    PT = PT_ref[...]

    for s in range(_UNROLL):
        t = x_ref[:, s * _WL:(s + 1) * _WL, :]
        o_ref[:, s * _WL:(s + 1) * _WL, :] = _fit_stripe(
            t, mn, sc, isc, lsb, ilsb, S, SQ, E, PT)


def _constants():
    w = jnp.arange(_W, dtype=jnp.int32)
    b = jnp.arange(_NB, dtype=jnp.int32)
    S = (w[:, None] // _WL == b[None, :]).astype(jnp.float32)
    SQ = S * (w[:, None] % _WL != _WL - 1).astype(jnp.float32)
    E = S.T
    r = jnp.arange(_R, dtype=jnp.int32)
    c = jnp.arange(_C, dtype=jnp.int32)
    PT = (r[None, :] // _WL == c[:, None]).astype(jnp.float32)
    return S, SQ, E, PT


def kernel(x):
    x2 = x[0]

    mx, mn = pl.pallas_call(
        _minmax_body,
        grid=(8,),
        in_specs=[pl.BlockSpec((_C, _H // 8, _W), lambda i: (0, i, 0))],
        out_specs=[
            pl.BlockSpec((1, 1), lambda i: (0, 0), memory_space=pltpu.SMEM),
            pl.BlockSpec((1, 1), lambda i: (0, 0), memory_space=pltpu.SMEM),
        ],
        out_shape=[
            jax.ShapeDtypeStruct((1, 1), jnp.float32),
            jax.ShapeDtypeStruct((1, 1), jnp.float32),
        ],
    )(x2)
    mx = mx[0, 0]
    mn = mn[0, 0]

    scale = (2.0 ** 16 - 1.0) / (mx - mn)
    iscale = 1.0 / scale
    lsb = 2.0 ** (jnp.round(jnp.log2(mx / 2.0 ** 15)) + 1.0)
    ilsb = 1.0 / lsb
    scalars = jnp.stack([mn, scale, iscale, lsb, ilsb,
                         jnp.float32(0), jnp.float32(0), jnp.float32(0)])
    scalars = scalars.astype(jnp.float32).reshape(1, 8)

    S, SQ, E, PT = _constants()
    inv = lambda i: (0, 0)

    out = pl.pallas_call(
        _fit_body,
        grid=(_H // (_WL * _UNROLL),),
        in_specs=[
            pl.BlockSpec(memory_space=pltpu.SMEM),
            pl.BlockSpec((_C, _WL * _UNROLL, _W), lambda i: (0, i, 0)),
            pl.BlockSpec((_W, _NB), inv),
            pl.BlockSpec((_W, _NB), inv),
            pl.BlockSpec((_NB, _W), inv),
            pl.BlockSpec((_C, _R), inv),
        ],
        out_specs=pl.BlockSpec((_C, _WL * _UNROLL, _W), lambda i: (0, i, 0)),
        out_shape=jax.ShapeDtypeStruct((_C, _H, _W), jnp.float32),
    )(scalars, x2, S, SQ, E, PT)
    return out[None]


# speedup vs baseline: 1.5074x; 1.1037x over previous
import jax
import jax.numpy as jnp
from jax.experimental import pallas as pl
from jax.experimental.pallas import tpu as pltpu

_WL = 8
_LOSS_THR = 1.0
_C = 32
_H = 512
_W = 512
_NB = _W // _WL
_R = _C * _WL


def _minmax_body(x_ref, mx_ref, mn_ref):
    i = pl.program_id(0)
    t = x_ref[...]
    xl = jnp.concatenate(
        [jnp.zeros((t.shape[0], t.shape[1], 1), t.dtype), t[:, :, :-1]], axis=2)
    xd = t - xl
    m = jnp.max(xd)
    n = jnp.min(xd)

    @pl.when(i == 0)
    def _():
        mx_ref[0, 0] = m
        mn_ref[0, 0] = n

    @pl.when(i > 0)
    def _():
        mx_ref[0, 0] = jnp.maximum(mx_ref[0, 0], m)
        mn_ref[0, 0] = jnp.minimum(mn_ref[0, 0], n)


_UNROLL = 4


def _fit_stripe(t, mn, sc, isc, lsb, ilsb, S, SQ, E, PT):
    xl = jnp.concatenate(
        [jnp.zeros((_C, _WL, 1), t.dtype), t[:, :, :-1]], axis=2)
    xd = t - xl
    v = jnp.round((xd - mn) * sc) * isc + mn

    a1 = v[0]
    a2 = v[1]

    def bsum(z):
        p = jnp.dot(z.reshape(_R, _W), S, preferred_element_type=jnp.float32)
        return p.reshape(_C, _WL, _NB).sum(axis=1)

    B3 = bsum(v)
    B1 = bsum(v * a1[None])
    B2 = bsum(v * a2[None])
    s1 = B3[0]
    s2 = B3[1]
    s11 = B1[0]
    s12 = B1[1]
    s22 = B2[1]
    n = jnp.float32(_WL * _WL)

    det = (s11 * (s22 * n - s2 * s2)
           - s12 * (s12 * n - s2 * s1)
           + s1 * (s12 * s2 - s22 * s1))
    sing = det == 0.0
    idet = 1.0 / jnp.where(sing, 1.0, det)
    i00 = jnp.where(sing, 1.0, (s22 * n - s2 * s2) * idet)
    i01 = jnp.where(sing, 0.0, (s1 * s2 - s12 * n) * idet)
    i02 = jnp.where(sing, 0.0, (s12 * s2 - s22 * s1) * idet)
    i11 = jnp.where(sing, 1.0, (s11 * n - s1 * s1) * idet)
    i12 = jnp.where(sing, 0.0, (s12 * s1 - s11 * s2) * idet)
    i22 = jnp.where(sing, 1.0, (s11 * s22 - s12 * s12) * idet)

    def san(c):
        return jnp.nan_to_num(c, nan=1e15, posinf=1e15, neginf=-1e15)

    c0 = san(i00[None] * B1 + i01[None] * B2 + i02[None] * B3)
    c1 = san(i01[None] * B1 + i11[None] * B2 + i12[None] * B3)
    c2 = san(i02[None] * B1 + i12[None] * B2 + i22[None] * B3)

    C0 = jnp.dot(c0, E, preferred_element_type=jnp.float32)
    C1 = jnp.dot(c1, E, preferred_element_type=jnp.float32)
    C2 = jnp.dot(c2, E, preferred_element_type=jnp.float32)

    r = C0[:, None, :] * a1[None] + C1[:, None, :] * a2[None] + C2[:, None, :]
    r1 = jnp.round(r * ilsb) * lsb

    sq = (v - r1) ** 2
    L = bsum(sq)

    q = (jnp.abs(a1 - jnp.roll(a1, -1, axis=1))
         + jnp.abs(a2 - jnp.roll(a2, -1, axis=1)))
    Q = jnp.dot(q.sum(axis=0)[None, :], SQ,
                preferred_element_type=jnp.float32)
    L = jnp.where(Q == 0.0, _LOSS_THR + 1.0, L)

    LE = jnp.dot(L, E, preferred_element_type=jnp.float32)
    take_fit = LE <= _LOSS_THR
    rr = jnp.where(take_fit[:, None, :], r1, v)
    return rr + xl


def _fit_body(s_ref, x_ref, S_ref, SQ_ref, E_ref, PT_ref, o_ref):
    mn = s_ref[0, 0]
    sc = s_ref[0, 1]
    isc = s_ref[0, 2]
    lsb = s_ref[0, 3]
    ilsb = s_ref[0, 4]
    S = S_ref[...]
    SQ = SQ_ref[...]
    E = E_ref[...]
    PT = PT_ref[...]

    for s in range(_UNROLL):
        t = x_ref[:, s * _WL:(s + 1) * _WL, :]
        o_ref[:, s * _WL:(s + 1) * _WL, :] = _fit_stripe(
            t, mn, sc, isc, lsb, ilsb, S, SQ, E, PT)


def _constants():
    w = jnp.arange(_W, dtype=jnp.int32)
    b = jnp.arange(_NB, dtype=jnp.int32)
    S = (w[:, None] // _WL == b[None, :]).astype(jnp.float32)
    SQ = S * (w[:, None] % _WL != _WL - 1).astype(jnp.float32)
    E = S.T
    r = jnp.arange(_R, dtype=jnp.int32)
    c = jnp.arange(_C, dtype=jnp.int32)
    PT = (r[None, :] // _WL == c[:, None]).astype(jnp.float32)
    return S, SQ, E, PT


def kernel(x):
    x2 = x[0]

    mx, mn = pl.pallas_call(
        _minmax_body,
        grid=(8,),
        in_specs=[pl.BlockSpec((_C, _H // 8, _W), lambda i: (0, i, 0))],
        out_specs=[
            pl.BlockSpec((1, 1), lambda i: (0, 0), memory_space=pltpu.SMEM),
            pl.BlockSpec((1, 1), lambda i: (0, 0), memory_space=pltpu.SMEM),
        ],
        out_shape=[
            jax.ShapeDtypeStruct((1, 1), jnp.float32),
            jax.ShapeDtypeStruct((1, 1), jnp.float32),
        ],
    )(x2)
    mx = mx[0, 0]
    mn = mn[0, 0]

    scale = (2.0 ** 16 - 1.0) / (mx - mn)
    iscale = 1.0 / scale
    lsb = 2.0 ** (jnp.round(jnp.log2(mx / 2.0 ** 15)) + 1.0)
    ilsb = 1.0 / lsb
    scalars = jnp.stack([mn, scale, iscale, lsb, ilsb,
                         jnp.float32(0), jnp.float32(0), jnp.float32(0)])
    scalars = scalars.astype(jnp.float32).reshape(1, 8)

    S, SQ, E, PT = _constants()
    inv = lambda i: (0, 0)

    out = pl.pallas_call(
        _fit_body,
        grid=(_H // (_WL * _UNROLL),),
        in_specs=[
            pl.BlockSpec(memory_space=pltpu.SMEM),
            pl.BlockSpec((_C, _WL * _UNROLL, _W), lambda i: (0, i, 0)),
            pl.BlockSpec((_W, _NB), inv),
            pl.BlockSpec((_W, _NB), inv),
            pl.BlockSpec((_NB, _W), inv),
            pl.BlockSpec((_C, _R), inv),
        ],
        out_specs=pl.BlockSpec((_C, _WL * _UNROLL, _W), lambda i: (0, i, 0)),
        out_shape=jax.ShapeDtypeStruct((_C, _H, _W), jnp.float32),
    )(scalars, x2, S, SQ, E, PT)
    return out[None]
